# baseline (speedup 1.0000x reference)
import functools

import jax
import jax.numpy as jnp
import numpy as np
from jax import lax
from jax.experimental import pallas as pl
from jax.experimental.pallas import tpu as pltpu
from jax.experimental.pallas import tpu_sc as plsc

N_NODES = 10000
NPAD = 10240
D = 128
NC = 2
NS = 16
NW = NC * NS
CH = 64
NBI = 4
NBF = 2
G = 40
NG = 4
CHUNKS = G * NG
EPW = CH * CHUNKS
E_PAD = NW * EPW
ROWS_PER_TILE = NPAD // NS
LANES = 16
ZCOPY = 128

_Q = np.arange(D)
_STORE_PERM = 16 * (2 * (_Q // 32) + (_Q % 2)) + ((_Q % 32) // 2)


def _scale_rows(ibuf, fbuf, val_v, c):
    def sgroup(gi, _):
        vals = val_v[c, pl.ds(gi * LANES, LANES)]
        for k in range(LANES):
            vv = jnp.full((LANES,), vals[k], dtype=jnp.float32)
            r = gi * LANES + k
            for w in range(D // 32):
                x = ibuf[r, pl.ds(LANES * w, LANES)]
                a = lax.bitcast_convert_type(x << 16, jnp.float32)
                b = lax.bitcast_convert_type(x & jnp.int32(-65536), jnp.float32)
                fbuf[r, pl.ds(32 * w, LANES)] = a * vv
                fbuf[r, pl.ds(32 * w + LANES, LANES)] = b * vv
        return 0

    lax.fori_loop(0, CH // LANES, sgroup, 0)


def _make_spmm():
    mesh = plsc.VectorSubcoreMesh(
        core_axis_name="c", subcore_axis_name="s", num_cores=NC, num_subcores=NS
    )

    @functools.partial(
        pl.kernel,
        out_type=jax.ShapeDtypeStruct((NC, NPAD, D), jnp.float32),
        mesh=mesh,
        compiler_params=pltpu.CompilerParams(use_tc_tiling_on_sc=False),
        scratch_types=[
            pltpu.VMEM((G, CH), jnp.int32),
            pltpu.VMEM((G, CH), jnp.int32),
            pltpu.VMEM((G, CH), jnp.float32),
            [pltpu.VMEM((CH, D // 2), jnp.int32)] * NBI,
            [pltpu.VMEM((CH, D), jnp.float32)] * NBF,
            [pltpu.SemaphoreType.DMA] * NBI,
            [pltpu.SemaphoreType.DMA] * NBF,
            pltpu.VMEM_SHARED((NPAD, D), jnp.float32),
        ],
    )
    def spmm(ego_hbm, src_hbm, dst_hbm, val_hbm, out_hbm,
             src_v, dst_v, val_v, ibufs, fbufs, gsem, ssem, acc):
        cid = lax.axis_index("c")
        sid = lax.axis_index("s")
        wid = cid * NS + sid

        zeros16 = jnp.zeros((LANES,), jnp.float32)

        def zrow0(r, _):
            for cc in range(D // LANES):
                fbufs[0][r, pl.ds(cc * LANES, LANES)] = zeros16
            return 0

        lax.fori_loop(0, CH, zrow0, 0)

        def zrow1(r, _):
            for cc in range(D // LANES):
                fbufs[1][r, pl.ds(cc * LANES, LANES)] = zeros16
            return 0

        lax.fori_loop(0, CH, zrow1, 0)

        base = sid * ROWS_PER_TILE
        for i in range(ROWS_PER_TILE // ZCOPY):
            pltpu.sync_copy(fbufs[0], acc.at[pl.ds(base + i * ZCOPY, CH)])
            pltpu.sync_copy(fbufs[1], acc.at[pl.ds(base + i * ZCOPY + CH, CH)])
        plsc.subcore_barrier()

        def gather(c, j):
            pltpu.async_copy(ego_hbm.at[src_v.at[c]], ibufs[j], gsem[j])

        def wait_gather(c, j):
            pltpu.make_async_copy(ego_hbm.at[src_v.at[c]], ibufs[j], gsem[j]).wait()

        def scatter(c, j):
            pltpu.async_copy(fbufs[j], acc.at[dst_v.at[c]], ssem[j], add=True)

        def wait_scatter(c, j):
            pltpu.make_async_copy(fbufs[j], acc.at[dst_v.at[c]], ssem[j]).wait()

        def group(g, _):
            @pl.when(g > 0)
            def _():
                wait_scatter(G - 2, 0)
                wait_scatter(G - 1, 1)

            sl = pl.ds(g * G, G)
            pltpu.sync_copy(src_hbm.at[wid, sl], src_v)
            pltpu.sync_copy(dst_hbm.at[wid, sl], dst_v)
            pltpu.sync_copy(val_hbm.at[wid, sl], val_v)

            for j in range(NBI):
                gather(j, j)

            def quad(i, _):
                c0 = i * NBI
                for jj in range(NBI):
                    c = c0 + jj
                    jf = jj % NBF
                    wait_gather(c, jj)
                    if jj >= NBF:
                        wait_scatter(c - NBF, jf)
                    else:
                        @pl.when(i > 0)
                        def _():
                            wait_scatter(c - NBF, jf)
                    _scale_rows(ibufs[jj], fbufs[jf], val_v, c)
                    scatter(c, jf)

                    @pl.when(c + NBI < G)
                    def _():
                        gather(c + NBI, jj)
                return 0

            lax.fori_loop(0, G // NBI, quad, 0)
            return 0

        lax.fori_loop(0, NG, group, 0)
        wait_scatter(G - 2, 0)
        wait_scatter(G - 1, 1)

        plsc.subcore_barrier()
        for i in range(ROWS_PER_TILE // ZCOPY):
            off = sid * ROWS_PER_TILE + i * ZCOPY
            pltpu.sync_copy(
                acc.at[pl.ds(off, ZCOPY)],
                out_hbm.at[cid, pl.ds(off, ZCOPY)],
            )

    return spmm


_spmm = _make_spmm()


def _tc_body(ego_ref, p0_ref, p1_ref, wt_ref, b_ref, out_ref):
    x = ego_ref[...] + p0_ref[0] + p1_ref[0]
    y = jnp.dot(x, wt_ref[...], preferred_element_type=jnp.float32)
    y = y + b_ref[...]
    out_ref[...] = jnp.where(y >= 0.0, y, y * 0.01)


def kernel(ego_embeddings, edge_index, edge_values, W, b):
    src = edge_index[0].astype(jnp.int32)
    dst = edge_index[1].astype(jnp.int32)
    e = src.shape[0]
    pad = E_PAD - e
    pad_src = jnp.arange(pad, dtype=jnp.int32) % N_NODES
    pad_dst = N_NODES + (jnp.arange(pad, dtype=jnp.int32) % (NPAD - N_NODES))
    src3 = jnp.concatenate([src, pad_src]).reshape(NW, CHUNKS, CH)
    dst3 = jnp.concatenate([dst, pad_dst]).reshape(NW, CHUNKS, CH)
    val3 = jnp.pad(edge_values, (0, pad)).reshape(NW, CHUNKS, CH)

    ego_bf = ego_embeddings[:, _STORE_PERM].astype(jnp.bfloat16)
    ego_pk = lax.bitcast_convert_type(
        ego_bf.reshape(N_NODES, D // 2, 2), jnp.int32
    )
    partials = _spmm(ego_pk, src3, dst3, val3)

    wt = W.T
    b2 = b.reshape(1, D)
    blk = 1000
    nblk = N_NODES // blk
    out = pl.pallas_call(
        _tc_body,
        grid=(nblk,),
        in_specs=[
            pl.BlockSpec((blk, D), lambda i: (i, 0)),
            pl.BlockSpec((1, blk, D), lambda i: (0, i, 0)),
            pl.BlockSpec((1, blk, D), lambda i: (1, i, 0)),
            pl.BlockSpec((D, D), lambda i: (0, 0)),
            pl.BlockSpec((1, D), lambda i: (0, 0)),
        ],
        out_specs=pl.BlockSpec((blk, D), lambda i: (i, 0)),
        out_shape=jax.ShapeDtypeStruct((N_NODES, D), jnp.float32),
    )(ego_embeddings, partials, partials, wt, b2)
    return out


# speedup vs baseline: 2.0799x; 2.0799x over previous
import functools

import jax
import jax.numpy as jnp
from jax import lax
from jax.experimental import pallas as pl
from jax.experimental.pallas import tpu as pltpu
from jax.experimental.pallas import tpu_sc as plsc

N_NODES = 10000
NPAD = 10240
D = 128
NC = 2
NS = 16
NW = NC * NS
CH = 64
NBUF = 4
G = 40
NG = 4
CHUNKS = G * NG
EPW = CH * CHUNKS
E_PAD = NW * EPW
ROWS_PER_TILE = NPAD // NS
LANES = 16
ZCOPY = 128


def _scale_rows(rows, val_v, c):
    def sgroup(g, _):
        vals = val_v[c, pl.ds(g * LANES, LANES)]
        for k in range(LANES):
            vv = jnp.full((LANES,), vals[k], dtype=jnp.float32)
            r = g * LANES + k
            for cc in range(D // LANES):
                sl = pl.ds(cc * LANES, LANES)
                rows[r, sl] = rows[r, sl] * vv
        return 0

    lax.fori_loop(0, CH // LANES, sgroup, 0)


def _make_spmm():
    mesh = plsc.VectorSubcoreMesh(
        core_axis_name="c", subcore_axis_name="s", num_cores=NC, num_subcores=NS
    )

    @functools.partial(
        pl.kernel,
        out_type=jax.ShapeDtypeStruct((NC, NPAD, D), jnp.float32),
        mesh=mesh,
        scratch_types=[
            pltpu.VMEM((G, CH), jnp.int32),
            pltpu.VMEM((G, CH), jnp.int32),
            pltpu.VMEM((G, CH), jnp.float32),
            [pltpu.VMEM((CH, D), jnp.float32)] * NBUF,
            [pltpu.SemaphoreType.DMA] * NBUF,
            [pltpu.SemaphoreType.DMA] * NBUF,
            pltpu.VMEM_SHARED((NPAD, D), jnp.float32),
        ],
    )
    def spmm(ego_hbm, src_hbm, dst_hbm, val_hbm, out_hbm,
             src_v, dst_v, val_v, rows, gsem, ssem, acc):
        cid = lax.axis_index("c")
        sid = lax.axis_index("s")
        wid = cid * NS + sid

        zeros16 = jnp.zeros((LANES,), jnp.float32)

        def zrow(r, _):
            for cc in range(D // LANES):
                rows[0][r, pl.ds(cc * LANES, LANES)] = zeros16
            return 0

        lax.fori_loop(0, min(ZCOPY, CH), zrow, 0)

        def zrow2(r, _):
            for cc in range(D // LANES):
                rows[1][r, pl.ds(cc * LANES, LANES)] = zeros16
            return 0

        lax.fori_loop(0, min(ZCOPY, CH), zrow2, 0)

        base = sid * ROWS_PER_TILE
        for i in range(ROWS_PER_TILE // ZCOPY):
            pltpu.sync_copy(rows[0], acc.at[pl.ds(base + i * ZCOPY, CH)])
            pltpu.sync_copy(rows[1], acc.at[pl.ds(base + i * ZCOPY + CH, CH)])
        plsc.subcore_barrier()

        def gather(c, j):
            pltpu.async_copy(ego_hbm.at[src_v.at[c]], rows[j], gsem[j])

        def wait_gather(c, j):
            pltpu.make_async_copy(ego_hbm.at[src_v.at[c]], rows[j], gsem[j]).wait()

        def scatter(c, j):
            pltpu.async_copy(rows[j], acc.at[dst_v.at[c]], ssem[j], add=True)

        def wait_scatter(c, j):
            pltpu.make_async_copy(rows[j], acc.at[dst_v.at[c]], ssem[j]).wait()

        def group(g, _):
            @pl.when(g > 0)
            def _():
                wait_scatter(G - 1, NBUF - 1)

            sl = pl.ds(g * G, G)
            pltpu.sync_copy(src_hbm.at[wid, sl], src_v)
            pltpu.sync_copy(dst_hbm.at[wid, sl], dst_v)
            pltpu.sync_copy(val_hbm.at[wid, sl], val_v)

            for j in range(NBUF - 1):
                gather(j, j)

            def quad(i, _):
                c0 = i * NBUF
                for jj in range(NBUF):
                    c = c0 + jj
                    jp = (jj + NBUF - 1) % NBUF
                    wait_gather(c, jj)
                    _scale_rows(rows[jj], val_v, c)
                    scatter(c, jj)
                    if jj == 0:
                        @pl.when(i > 0)
                        def _():
                            wait_scatter(c - 1, jp)
                    else:
                        wait_scatter(c - 1, jp)

                    @pl.when(c + NBUF - 1 < G)
                    def _():
                        gather(c + NBUF - 1, jp)
                return 0

            lax.fori_loop(0, G // NBUF, quad, 0)
            return 0

        lax.fori_loop(0, NG, group, 0)
        wait_scatter(G - 1, NBUF - 1)

        plsc.subcore_barrier()
        for i in range(ROWS_PER_TILE // ZCOPY):
            off = sid * ROWS_PER_TILE + i * ZCOPY
            pltpu.sync_copy(
                acc.at[pl.ds(off, ZCOPY)],
                out_hbm.at[cid, pl.ds(off, ZCOPY)],
            )

    return spmm


_spmm = _make_spmm()


def _tc_body(ego_ref, p0_ref, p1_ref, wt_ref, b_ref, out_ref):
    x = ego_ref[...] + p0_ref[0] + p1_ref[0]
    y = jnp.dot(x, wt_ref[...], preferred_element_type=jnp.float32)
    y = y + b_ref[...]
    out_ref[...] = jnp.where(y >= 0.0, y, y * 0.01)


def kernel(ego_embeddings, edge_index, edge_values, W, b):
    src = edge_index[0].astype(jnp.int32)
    dst = edge_index[1].astype(jnp.int32)
    e = src.shape[0]
    pad = E_PAD - e
    pad_src = jnp.arange(pad, dtype=jnp.int32) % N_NODES
    pad_dst = N_NODES + (jnp.arange(pad, dtype=jnp.int32) % (NPAD - N_NODES))
    src3 = jnp.concatenate([src, pad_src]).reshape(NW, CHUNKS, CH)
    dst3 = jnp.concatenate([dst, pad_dst]).reshape(NW, CHUNKS, CH)
    val3 = jnp.pad(edge_values, (0, pad)).reshape(NW, CHUNKS, CH)

    partials = _spmm(ego_embeddings, src3, dst3, val3)

    wt = W.T
    b2 = b.reshape(1, D)
    blk = 1000
    nblk = N_NODES // blk
    out = pl.pallas_call(
        _tc_body,
        grid=(nblk,),
        in_specs=[
            pl.BlockSpec((blk, D), lambda i: (i, 0)),
            pl.BlockSpec((1, blk, D), lambda i: (0, i, 0)),
            pl.BlockSpec((1, blk, D), lambda i: (1, i, 0)),
            pl.BlockSpec((D, D), lambda i: (0, 0)),
            pl.BlockSpec((1, D), lambda i: (0, 0)),
        ],
        out_specs=pl.BlockSpec((blk, D), lambda i: (i, 0)),
        out_shape=jax.ShapeDtypeStruct((N_NODES, D), jnp.float32),
    )(ego_embeddings, partials, partials, wt, b2)
    return out
